# baseline (speedup 1.0000x reference)
import jax
import jax.numpy as jnp
from jax.experimental import pallas as pl


_ROWS = 4096
_SEQ = 50
_NUM = 256
_TOTAL = _ROWS * _SEQ
_BLOCK_M = 2048
_GRID = _TOTAL // _BLOCK_M


def _onehot_block(idx_ref, out_ref):
    idx = idx_ref[0, 0, :]
    iota = jax.lax.broadcasted_iota(jnp.int32, (_BLOCK_M, _NUM), 1)
    out_ref[...] = (idx[:, None] == iota).astype(jnp.float32)


def kernel(input_tensor):
    idx = input_tensor.reshape(_GRID, 1, _BLOCK_M)
    flat = pl.pallas_call(
        _onehot_block,
        grid=(_GRID,),
        in_specs=[pl.BlockSpec((1, 1, _BLOCK_M), lambda i: (i, 0, 0))],
        out_specs=pl.BlockSpec((_BLOCK_M, _NUM), lambda i: (i, 0)),
        out_shape=jax.ShapeDtypeStruct((_TOTAL, _NUM), jnp.float32),
    )(idx)
    return flat.reshape(_ROWS, _SEQ, _NUM)


# speedup vs baseline: 2.1851x; 2.1851x over previous
import jax
import jax.numpy as jnp
from jax.experimental import pallas as pl


_ROWS = 4096
_SEQ = 50
_NUM = 256
_TOTAL = _ROWS * _SEQ
_BLOCK_M = 2048
_GRID = _TOTAL // _BLOCK_M


_BLOCK_R = 256


def _onehot_block(idx_ref, out_ref):
    idx = idx_ref[...]
    iota = jax.lax.broadcasted_iota(jnp.int32, (_BLOCK_R, _SEQ, _NUM), 2)
    out_ref[...] = (idx[:, :, None] == iota).astype(jnp.float32)


def kernel(input_tensor):
    return pl.pallas_call(
        _onehot_block,
        grid=(_ROWS // _BLOCK_R,),
        in_specs=[pl.BlockSpec((_BLOCK_R, _SEQ), lambda i: (i, 0))],
        out_specs=pl.BlockSpec((_BLOCK_R, _SEQ, _NUM), lambda i: (i, 0, 0)),
        out_shape=jax.ShapeDtypeStruct((_ROWS, _SEQ, _NUM), jnp.float32),
    )(input_tensor)


# speedup vs baseline: 2.1990x; 1.0064x over previous
import jax
import jax.numpy as jnp
from jax.experimental import pallas as pl
from jax.experimental.pallas import tpu as pltpu


_ROWS = 4096
_SEQ = 50
_NUM = 256
_CHUNK_R = 128
_NCHUNK = _ROWS // _CHUNK_R
_K = 4


def _onehot_manual(idx_ref, out_ref, buf, sem):
    for i in range(_NCHUNK):
        slot = i % _K
        if i >= _K:
            pltpu.make_async_copy(
                buf.at[slot],
                out_ref.at[pl.ds((i - _K) * _CHUNK_R, _CHUNK_R)],
                sem.at[slot],
            ).wait()
        idx = idx_ref[pl.ds(i * _CHUNK_R, _CHUNK_R), :]
        iota = jax.lax.broadcasted_iota(jnp.int32, (_CHUNK_R, _SEQ, _NUM), 2)
        buf[slot] = (idx[:, :, None] == iota).astype(jnp.float32)
        pltpu.make_async_copy(
            buf.at[slot],
            out_ref.at[pl.ds(i * _CHUNK_R, _CHUNK_R)],
            sem.at[slot],
        ).start()
    for i in range(_NCHUNK - _K, _NCHUNK):
        slot = i % _K
        pltpu.make_async_copy(
            buf.at[slot],
            out_ref.at[pl.ds(i * _CHUNK_R, _CHUNK_R)],
            sem.at[slot],
        ).wait()


def kernel(input_tensor):
    return pl.pallas_call(
        _onehot_manual,
        in_specs=[pl.BlockSpec(memory_space=pltpu.VMEM)],
        out_specs=pl.BlockSpec(memory_space=pltpu.HBM),
        out_shape=jax.ShapeDtypeStruct((_ROWS, _SEQ, _NUM), jnp.float32),
        scratch_shapes=[
            pltpu.VMEM((_K, _CHUNK_R, _SEQ, _NUM), jnp.float32),
            pltpu.SemaphoreType.DMA((_K,)),
        ],
    )(input_tensor)


# speedup vs baseline: 2.2263x; 1.0124x over previous
import jax
import jax.numpy as jnp
from jax.experimental import pallas as pl
from jax.experimental.pallas import tpu as pltpu


_ROWS = 4096
_SEQ = 50
_NUM = 256
_CHUNK_R = 128
_NCHUNK = _ROWS // _CHUNK_R
_K = 4


def _onehot_manual(idx_ref, out_ref, buf, sem):
    for i in range(_NCHUNK):
        slot = i % _K
        if i >= _K:
            pltpu.make_async_copy(
                buf.at[slot],
                out_ref.at[pl.ds((i - _K) * _CHUNK_R, _CHUNK_R)],
                sem.at[slot],
            ).wait()
        idx = idx_ref[pl.ds(i * _CHUNK_R, _CHUNK_R), :]
        iota = jax.lax.broadcasted_iota(jnp.int32, (_CHUNK_R, _SEQ, _NUM), 2)
        buf[slot] = (idx[:, :, None] == iota).astype(jnp.float32)
        pltpu.make_async_copy(
            buf.at[slot],
            out_ref.at[pl.ds(i * _CHUNK_R, _CHUNK_R)],
            sem.at[slot],
        ).start(priority=slot % 2)
    for i in range(_NCHUNK - _K, _NCHUNK):
        slot = i % _K
        pltpu.make_async_copy(
            buf.at[slot],
            out_ref.at[pl.ds(i * _CHUNK_R, _CHUNK_R)],
            sem.at[slot],
        ).wait()


def kernel(input_tensor):
    return pl.pallas_call(
        _onehot_manual,
        in_specs=[pl.BlockSpec(memory_space=pltpu.VMEM)],
        out_specs=pl.BlockSpec(memory_space=pltpu.HBM),
        out_shape=jax.ShapeDtypeStruct((_ROWS, _SEQ, _NUM), jnp.float32),
        scratch_shapes=[
            pltpu.VMEM((_K, _CHUNK_R, _SEQ, _NUM), jnp.float32),
            pltpu.SemaphoreType.DMA((_K,)),
        ],
    )(input_tensor)


# speedup vs baseline: 7.0292x; 3.1573x over previous
import jax
import jax.numpy as jnp
from jax.experimental import pallas as pl


_ROWS = 4096
_SEQ = 50
_NUM = 256
_BLOCK_R = 128


def _onehot_block(idx_ref, out_ref):
    idx = idx_ref[...]
    iota = jax.lax.broadcasted_iota(jnp.int32, (_SEQ, _BLOCK_R, _NUM), 2)
    out_ref[...] = (idx[:, :, None] == iota).astype(jnp.float32)


def kernel(input_tensor):
    idx_t = input_tensor.T
    out_t = pl.pallas_call(
        _onehot_block,
        grid=(_ROWS // _BLOCK_R,),
        in_specs=[pl.BlockSpec((_SEQ, _BLOCK_R), lambda i: (0, i))],
        out_specs=pl.BlockSpec((_SEQ, _BLOCK_R, _NUM), lambda i: (0, i, 0)),
        out_shape=jax.ShapeDtypeStruct((_SEQ, _ROWS, _NUM), jnp.float32),
    )(idx_t)
    return out_t.transpose(1, 0, 2)


# speedup vs baseline: 7.1320x; 1.0146x over previous
import jax
import jax.numpy as jnp
from jax.experimental import pallas as pl


_ROWS = 4096
_SEQ = 50
_NUM = 256
_BLOCK_R = 256


def _onehot_block(idx_ref, out_ref):
    idx = idx_ref[...]
    iota = jax.lax.broadcasted_iota(jnp.int32, (_SEQ, _BLOCK_R, _NUM), 2)
    out_ref[...] = (idx[:, :, None] == iota).astype(jnp.float32)


def kernel(input_tensor):
    idx_t = input_tensor.T
    out_t = pl.pallas_call(
        _onehot_block,
        grid=(_ROWS // _BLOCK_R,),
        in_specs=[pl.BlockSpec((_SEQ, _BLOCK_R), lambda i: (0, i))],
        out_specs=pl.BlockSpec((_SEQ, _BLOCK_R, _NUM), lambda i: (0, i, 0)),
        out_shape=jax.ShapeDtypeStruct((_SEQ, _ROWS, _NUM), jnp.float32),
    )(idx_t)
    return out_t.transpose(1, 0, 2)
